# Initial kernel scaffold; baseline (speedup 1.0000x reference)
import jax
import jax.numpy as jnp
from jax.experimental import pallas as pl


def kernel(x, edge_index, edge_attr, params):
    raise NotImplementedError("write your pallas kernel here")


# speedup vs baseline: 3.1234x; 3.1234x over previous
import functools

import jax
import jax.numpy as jnp
from jax import lax
from jax.experimental import pallas as pl
from jax.experimental.pallas import tpu as pltpu
from jax.experimental.pallas import tpu_sc as plsc

_F32 = jnp.float32
_HID = 128
_NN = 10000
_NE = 320000
_DEPTH = 3

_EBLK = 2000
_NBLK = 2000
_NC = 2
_NS = 16
_EPW = _NE // (_NC * _NS)
_C = 80
_NCH = _EPW // _C


def _relu(v):
    return jnp.maximum(v, 0.0)


def _mm(a, w):
    return lax.dot_general(a, w, (((1,), (0,)), ((), ())),
                           preferred_element_type=_F32)


def _edge_call(e_or_ea, gi, gj, ws, bs, first, wei=None, bei=None):
    def body(*refs):
        if first:
            (ea_ref, gi_ref, gj_ref, wei_ref, bei_ref, w_ref, b_ref,
             enew_ref, oe_ref) = refs
            e = _mm(ea_ref[...], wei_ref[...]) + bei_ref[...]
        else:
            e_ref, gi_ref, gj_ref, w_ref, b_ref, enew_ref, oe_ref = refs
            e = e_ref[...]
        f = _mm(e, w_ref[0]) + b_ref[0]
        out = _relu(f + gi_ref[...] + gj_ref[...])
        t = _relu(_mm(out, w_ref[1]) + b_ref[1])
        t = _relu(_mm(t, w_ref[2]) + b_ref[2])
        en = e + (_mm(t, w_ref[3]) + b_ref[3])
        enew_ref[...] = en
        u = _relu(_mm(en, w_ref[4]) + b_ref[4])
        u = _relu(_mm(u, w_ref[5]) + b_ref[5])
        oe_ref[...] = _mm(u, w_ref[6]) + b_ref[6]

    din = e_or_ea.shape[1]
    in_specs = [
        pl.BlockSpec((_EBLK, din), lambda i: (i, 0)),
        pl.BlockSpec((_EBLK, _HID), lambda i: (i, 0)),
        pl.BlockSpec((_EBLK, _HID), lambda i: (i, 0)),
    ]
    args = [e_or_ea, gi, gj]
    if first:
        in_specs += [pl.BlockSpec((din, _HID), lambda i: (0, 0)),
                     pl.BlockSpec((1, _HID), lambda i: (0, 0))]
        args += [wei, bei]
    in_specs += [pl.BlockSpec((7, _HID, _HID), lambda i: (0, 0, 0)),
                 pl.BlockSpec((7, 1, _HID), lambda i: (0, 0, 0))]
    args += [ws, bs]
    return pl.pallas_call(
        body,
        grid=(_NE // _EBLK,),
        in_specs=in_specs,
        out_specs=[pl.BlockSpec((_EBLK, _HID), lambda i: (i, 0))] * 2,
        out_shape=[jax.ShapeDtypeStruct((_NE, _HID), _F32)] * 2,
    )(*args)


def _node_call(h, p0, p1, ws, bs, wij, bij, want_f):
    def body(*refs):
        if want_f:
            (h_ref, p0_ref, p1_ref, w_ref, b_ref, wij_ref, bij_ref,
             hn_ref, fi_ref, fj_ref) = refs
        else:
            h_ref, p0_ref, p1_ref, w_ref, b_ref, hn_ref = refs
        o = p0_ref[...] + p1_ref[...]
        t = _relu(_mm(o, w_ref[0]) + b_ref[0])
        t = _relu(_mm(t, w_ref[1]) + b_ref[1])
        hn = h_ref[...] + (_mm(t, w_ref[2]) + b_ref[2])
        hn_ref[...] = hn
        if want_f:
            fi_ref[...] = _mm(hn, wij_ref[0]) + bij_ref[0]
            fj_ref[...] = _mm(hn, wij_ref[1]) + bij_ref[1]

    blk = pl.BlockSpec((_NBLK, _HID), lambda i: (i, 0))
    in_specs = [blk, blk, blk,
                pl.BlockSpec((3, _HID, _HID), lambda i: (0, 0, 0)),
                pl.BlockSpec((3, 1, _HID), lambda i: (0, 0, 0))]
    args = [h, p0, p1, ws, bs]
    n_out = 1
    if want_f:
        in_specs += [pl.BlockSpec((2, _HID, _HID), lambda i: (0, 0, 0)),
                     pl.BlockSpec((2, 1, _HID), lambda i: (0, 0, 0))]
        args += [wij, bij]
        n_out = 3
    res = pl.pallas_call(
        body,
        grid=(_NN // _NBLK,),
        in_specs=in_specs,
        out_specs=[blk] * n_out,
        out_shape=[jax.ShapeDtypeStruct((_NN, _HID), _F32)] * n_out,
    )(*args)
    return res if want_f else res[0]


def _prologue_call(x, wn, bn, wij, bij):
    def body(x_ref, w_ref, b_ref, wij_ref, bij_ref, h_ref, fi_ref, fj_ref):
        hn = _mm(x_ref[...], w_ref[...]) + b_ref[...]
        h_ref[...] = hn
        fi_ref[...] = _mm(hn, wij_ref[0]) + bij_ref[0]
        fj_ref[...] = _mm(hn, wij_ref[1]) + bij_ref[1]

    blk = pl.BlockSpec((_NBLK, _HID), lambda i: (i, 0))
    return pl.pallas_call(
        body,
        grid=(_NN // _NBLK,),
        in_specs=[blk,
                  pl.BlockSpec((_HID, _HID), lambda i: (0, 0)),
                  pl.BlockSpec((1, _HID), lambda i: (0, 0)),
                  pl.BlockSpec((2, _HID, _HID), lambda i: (0, 0, 0)),
                  pl.BlockSpec((2, 1, _HID), lambda i: (0, 0, 0))],
        out_specs=[blk] * 3,
        out_shape=[jax.ShapeDtypeStruct((_NN, _HID), _F32)] * 3,
    )(x, wn, bn, wij, bij)


def _sc_mesh():
    return plsc.VectorSubcoreMesh(core_axis_name="c", subcore_axis_name="s",
                                  num_cores=_NC, num_subcores=_NS)


def _gather_call(fi, fj, row4, col4):
    @functools.partial(
        pl.kernel,
        out_type=[jax.ShapeDtypeStruct((_NE, _HID), _F32)] * 2,
        mesh=_sc_mesh(),
        scratch_types=[
            pltpu.VMEM((_NCH, _C), jnp.int32),
            pltpu.VMEM((_NCH, _C), jnp.int32),
            pltpu.VMEM((_C, _HID), _F32),
            pltpu.VMEM((_C, _HID), _F32),
            pltpu.SemaphoreType.DMA,
            pltpu.SemaphoreType.DMA,
        ],
    )
    def k(fi_hbm, fj_hbm, row_hbm, col_hbm, gi_hbm, gj_hbm,
          idxi, idxj, bufi, bufj, semi, semj):
        cid = lax.axis_index("c")
        sid = lax.axis_index("s")
        base = (cid * _NS + sid) * _EPW
        pltpu.sync_copy(row_hbm.at[cid, sid], idxi)
        pltpu.sync_copy(col_hbm.at[cid, sid], idxj)

        def body(it, carry):
            ci = pltpu.async_copy(fi_hbm.at[idxi.at[it]], bufi, semi)
            cj = pltpu.async_copy(fj_hbm.at[idxj.at[it]], bufj, semj)
            off = base + it * _C
            ci.wait()
            pltpu.sync_copy(bufi, gi_hbm.at[pl.ds(off, _C)])
            cj.wait()
            pltpu.sync_copy(bufj, gj_hbm.at[pl.ds(off, _C)])
            return carry

        lax.fori_loop(0, _NCH, body, 0)

    return k(fi, fj, row4, col4)


def _scatter_call(oe, col4, zeros):
    @functools.partial(
        pl.kernel,
        out_type=jax.ShapeDtypeStruct((_NC, _NN, _HID), _F32),
        mesh=_sc_mesh(),
        scratch_types=[
            pltpu.VMEM((_NCH, _C), jnp.int32),
            pltpu.VMEM((_C, _HID), _F32),
            pltpu.VMEM_SHARED((_NN, _HID), _F32),
        ],
    )
    def k(oe_hbm, col_hbm, z_hbm, out_hbm, idx, buf, acc):
        cid = lax.axis_index("c")
        sid = lax.axis_index("s")

        @pl.when(sid == 0)
        def _zero():
            pltpu.sync_copy(z_hbm, acc)

        plsc.subcore_barrier()
        pltpu.sync_copy(col_hbm.at[cid, sid], idx)
        base = (cid * _NS + sid) * _EPW

        def body(it, carry):
            off = base + it * _C
            pltpu.sync_copy(oe_hbm.at[pl.ds(off, _C)], buf)
            pltpu.sync_copy(buf, acc.at[idx.at[it]], add=True)
            return carry

        lax.fori_loop(0, _NCH, body, 0)
        plsc.subcore_barrier()

        @pl.when(sid == 0)
        def _writeout():
            pltpu.sync_copy(acc, out_hbm.at[cid])

    return k(oe, col4, zeros)


def kernel(x, edge_index, edge_attr, params):
    row4 = edge_index[0].reshape(_NC, _NS, _NCH, _C)
    col4 = edge_index[1].reshape(_NC, _NS, _NCH, _C)

    em = params["edge_model"]
    nm = params["node_model"]
    w_edge = jnp.stack([em["edge"]["w"]]
                       + [l["w"] for l in em["mlp"]["lins"]]
                       + [l["w"] for l in nm["mlp1"]["lins"]])
    b_edge = jnp.stack([em["edge"]["b"]]
                       + [l["b"] for l in em["mlp"]["lins"]]
                       + [l["b"] for l in nm["mlp1"]["lins"]]).reshape(7, 1, _HID)
    w_node = jnp.stack([l["w"] for l in nm["mlp2"]["lins"]])
    b_node = jnp.stack([l["b"] for l in nm["mlp2"]["lins"]]).reshape(3, 1, _HID)
    wij = jnp.stack([em["node_in"]["w"], em["node_out"]["w"]])
    bij = jnp.stack([em["node_in"]["b"], em["node_out"]["b"]]).reshape(2, 1, _HID)
    wei = params["edge_init"]["w"]
    bei = params["edge_init"]["b"].reshape(1, _HID)
    wn = params["node_init"]["w"]
    bn = params["node_init"]["b"].reshape(1, _HID)
    zeros = jnp.zeros((_NN, _HID), _F32)

    h, fi, fj = _prologue_call(x, wn, bn, wij, bij)
    e = None
    for layer in range(_DEPTH):
        gi, gj = _gather_call(fi, fj, row4, col4)
        if layer == 0:
            e, oe = _edge_call(edge_attr, gi, gj, w_edge, b_edge, True, wei, bei)
        else:
            e, oe = _edge_call(e, gi, gj, w_edge, b_edge, False)
        p = _scatter_call(oe, col4, zeros)
        if layer == _DEPTH - 1:
            h = _node_call(h, p[0], p[1], w_node, b_node, None, None, False)
        else:
            h, fi, fj = _node_call(h, p[0], p[1], w_node, b_node, wij, bij, True)
    return (h, e)


# speedup vs baseline: 3.5201x; 1.1270x over previous
import functools

import jax
import jax.numpy as jnp
from jax import lax
from jax.experimental import pallas as pl
from jax.experimental.pallas import tpu as pltpu
from jax.experimental.pallas import tpu_sc as plsc

_F32 = jnp.float32
_HID = 128
_NN = 10000
_NE = 320000
_DEPTH = 3

_EBLK = 2000
_NBLK = 2000
_NC = 2
_NS = 16
_EPW = _NE // (_NC * _NS)
_C = 80
_NCH = _EPW // _C


def _relu(v):
    return jnp.maximum(v, 0.0)


def _mm(a, w):
    return lax.dot_general(a, w, (((1,), (0,)), ((), ())),
                           preferred_element_type=_F32)


def _edge_call(e_or_ea, gi, gj, ws, bs, first, wei=None, bei=None):
    def body(*refs):
        if first:
            (ea_ref, gi_ref, gj_ref, wei_ref, bei_ref, w_ref, b_ref,
             enew_ref, oe_ref) = refs
            e = _mm(ea_ref[...], wei_ref[...]) + bei_ref[...]
        else:
            e_ref, gi_ref, gj_ref, w_ref, b_ref, enew_ref, oe_ref = refs
            e = e_ref[...]
        f = _mm(e, w_ref[0]) + b_ref[0]
        out = _relu(f + gi_ref[...] + gj_ref[...])
        t = _relu(_mm(out, w_ref[1]) + b_ref[1])
        t = _relu(_mm(t, w_ref[2]) + b_ref[2])
        en = e + (_mm(t, w_ref[3]) + b_ref[3])
        enew_ref[...] = en
        u = _relu(_mm(en, w_ref[4]) + b_ref[4])
        u = _relu(_mm(u, w_ref[5]) + b_ref[5])
        oe_ref[...] = _mm(u, w_ref[6]) + b_ref[6]

    din = e_or_ea.shape[1]
    in_specs = [
        pl.BlockSpec((_EBLK, din), lambda i: (i, 0)),
        pl.BlockSpec((_EBLK, _HID), lambda i: (i, 0)),
        pl.BlockSpec((_EBLK, _HID), lambda i: (i, 0)),
    ]
    args = [e_or_ea, gi, gj]
    if first:
        in_specs += [pl.BlockSpec((din, _HID), lambda i: (0, 0)),
                     pl.BlockSpec((1, _HID), lambda i: (0, 0))]
        args += [wei, bei]
    in_specs += [pl.BlockSpec((7, _HID, _HID), lambda i: (0, 0, 0)),
                 pl.BlockSpec((7, 1, _HID), lambda i: (0, 0, 0))]
    args += [ws, bs]
    return pl.pallas_call(
        body,
        grid=(_NE // _EBLK,),
        in_specs=in_specs,
        out_specs=[pl.BlockSpec((_EBLK, _HID), lambda i: (i, 0))] * 2,
        out_shape=[jax.ShapeDtypeStruct((_NE, _HID), _F32)] * 2,
    )(*args)


def _node_call(h, p0, p1, ws, bs, wij, bij, want_f):
    def body(*refs):
        if want_f:
            (h_ref, p0_ref, p1_ref, w_ref, b_ref, wij_ref, bij_ref,
             hn_ref, fi_ref, fj_ref) = refs
        else:
            h_ref, p0_ref, p1_ref, w_ref, b_ref, hn_ref = refs
        o = p0_ref[...] + p1_ref[...]
        t = _relu(_mm(o, w_ref[0]) + b_ref[0])
        t = _relu(_mm(t, w_ref[1]) + b_ref[1])
        hn = h_ref[...] + (_mm(t, w_ref[2]) + b_ref[2])
        hn_ref[...] = hn
        if want_f:
            fi_ref[...] = _mm(hn, wij_ref[0]) + bij_ref[0]
            fj_ref[...] = _mm(hn, wij_ref[1]) + bij_ref[1]

    blk = pl.BlockSpec((_NBLK, _HID), lambda i: (i, 0))
    in_specs = [blk, blk, blk,
                pl.BlockSpec((3, _HID, _HID), lambda i: (0, 0, 0)),
                pl.BlockSpec((3, 1, _HID), lambda i: (0, 0, 0))]
    args = [h, p0, p1, ws, bs]
    n_out = 1
    if want_f:
        in_specs += [pl.BlockSpec((2, _HID, _HID), lambda i: (0, 0, 0)),
                     pl.BlockSpec((2, 1, _HID), lambda i: (0, 0, 0))]
        args += [wij, bij]
        n_out = 3
    res = pl.pallas_call(
        body,
        grid=(_NN // _NBLK,),
        in_specs=in_specs,
        out_specs=[blk] * n_out,
        out_shape=[jax.ShapeDtypeStruct((_NN, _HID), _F32)] * n_out,
    )(*args)
    return res if want_f else res[0]


def _prologue_call(x, wn, bn, wij, bij):
    def body(x_ref, w_ref, b_ref, wij_ref, bij_ref, h_ref, fi_ref, fj_ref):
        hn = _mm(x_ref[...], w_ref[...]) + b_ref[...]
        h_ref[...] = hn
        fi_ref[...] = _mm(hn, wij_ref[0]) + bij_ref[0]
        fj_ref[...] = _mm(hn, wij_ref[1]) + bij_ref[1]

    blk = pl.BlockSpec((_NBLK, _HID), lambda i: (i, 0))
    return pl.pallas_call(
        body,
        grid=(_NN // _NBLK,),
        in_specs=[blk,
                  pl.BlockSpec((_HID, _HID), lambda i: (0, 0)),
                  pl.BlockSpec((1, _HID), lambda i: (0, 0)),
                  pl.BlockSpec((2, _HID, _HID), lambda i: (0, 0, 0)),
                  pl.BlockSpec((2, 1, _HID), lambda i: (0, 0, 0))],
        out_specs=[blk] * 3,
        out_shape=[jax.ShapeDtypeStruct((_NN, _HID), _F32)] * 3,
    )(x, wn, bn, wij, bij)


def _sc_mesh():
    return plsc.VectorSubcoreMesh(core_axis_name="c", subcore_axis_name="s",
                                  num_cores=_NC, num_subcores=_NS)


def _gather_call(fi, fj, row4, col4):
    @functools.partial(
        pl.kernel,
        out_type=[jax.ShapeDtypeStruct((_NE, _HID), _F32)] * 2,
        mesh=_sc_mesh(),
        scratch_types=[
            pltpu.VMEM((_NCH, _C), jnp.int32),
            pltpu.VMEM((_NCH, _C), jnp.int32),
            [pltpu.VMEM((_C, _HID), _F32)] * 2,
            [pltpu.VMEM((_C, _HID), _F32)] * 2,
            [pltpu.SemaphoreType.DMA] * 2,
            [pltpu.SemaphoreType.DMA] * 2,
            [pltpu.SemaphoreType.DMA] * 2,
            [pltpu.SemaphoreType.DMA] * 2,
        ],
    )
    def k(fi_hbm, fj_hbm, row_hbm, col_hbm, gi_hbm, gj_hbm,
          idxi, idxj, bufi, bufj, gsi, gsj, wsi, wsj):
        cid = lax.axis_index("c")
        sid = lax.axis_index("s")
        base = (cid * _NS + sid) * _EPW
        pltpu.sync_copy(row_hbm.at[cid, sid], idxi)
        pltpu.sync_copy(col_hbm.at[cid, sid], idxj)

        def start_gather(c, b):
            pltpu.async_copy(fi_hbm.at[idxi.at[c]], bufi[b], gsi[b])
            pltpu.async_copy(fj_hbm.at[idxj.at[c]], bufj[b], gsj[b])

        def wait_gather(b):
            pltpu.make_async_copy(fi_hbm.at[idxi.at[0]], bufi[b], gsi[b]).wait()
            pltpu.make_async_copy(fj_hbm.at[idxj.at[0]], bufj[b], gsj[b]).wait()

        def start_write(c, b):
            off = base + c * _C
            pltpu.async_copy(bufi[b], gi_hbm.at[pl.ds(off, _C)], wsi[b])
            pltpu.async_copy(bufj[b], gj_hbm.at[pl.ds(off, _C)], wsj[b])

        def wait_write(b):
            pltpu.make_async_copy(bufi[b], gi_hbm.at[pl.ds(0, _C)], wsi[b]).wait()
            pltpu.make_async_copy(bufj[b], gj_hbm.at[pl.ds(0, _C)], wsj[b]).wait()

        start_gather(0, 0)
        wait_gather(0)
        start_write(0, 0)
        start_gather(1, 1)
        wait_gather(1)
        start_write(1, 1)
        wait_write(0)
        start_gather(2, 0)

        def body(p, carry):
            c = 2 * p
            wait_gather(0)
            start_write(c, 0)
            wait_write(1)
            start_gather(c + 1, 1)
            wait_gather(1)
            start_write(c + 1, 1)
            wait_write(0)
            start_gather(c + 2, 0)
            return carry

        lax.fori_loop(1, 62, body, 0)
        wait_gather(0)
        start_write(_NCH - 1, 0)
        wait_write(1)
        wait_write(0)

    return k(fi, fj, row4, col4)


def _scatter_call(oe, col4, zeros):
    @functools.partial(
        pl.kernel,
        out_type=jax.ShapeDtypeStruct((_NC, _NN, _HID), _F32),
        mesh=_sc_mesh(),
        scratch_types=[
            pltpu.VMEM((_NCH, _C), jnp.int32),
            [pltpu.VMEM((_C, _HID), _F32)] * 2,
            pltpu.VMEM_SHARED((_NN, _HID), _F32),
            [pltpu.SemaphoreType.DMA] * 2,
            [pltpu.SemaphoreType.DMA] * 2,
        ],
    )
    def k(oe_hbm, col_hbm, z_hbm, out_hbm, idx, buf, acc, lsem, ssem):
        cid = lax.axis_index("c")
        sid = lax.axis_index("s")

        @pl.when(sid == 0)
        def _zero():
            pltpu.sync_copy(z_hbm, acc)

        pltpu.sync_copy(col_hbm.at[cid, sid], idx)
        plsc.subcore_barrier()
        base = (cid * _NS + sid) * _EPW

        def start_load(c, b):
            off = base + c * _C
            pltpu.async_copy(oe_hbm.at[pl.ds(off, _C)], buf[b], lsem[b])

        def wait_load(b):
            pltpu.make_async_copy(oe_hbm.at[pl.ds(0, _C)], buf[b], lsem[b]).wait()

        def start_add(c, b):
            pltpu.async_copy(buf[b], acc.at[idx.at[c]], ssem[b], add=True)

        def wait_add(b):
            pltpu.make_async_copy(buf[b], acc.at[idx.at[0]], ssem[b]).wait()

        start_load(0, 0)
        wait_load(0)
        start_add(0, 0)
        start_load(1, 1)
        wait_load(1)
        start_add(1, 1)
        wait_add(0)
        start_load(2, 0)

        def body(p, carry):
            c = 2 * p
            wait_load(0)
            start_add(c, 0)
            wait_add(1)
            start_load(c + 1, 1)
            wait_load(1)
            start_add(c + 1, 1)
            wait_add(0)
            start_load(c + 2, 0)
            return carry

        lax.fori_loop(1, 62, body, 0)
        wait_load(0)
        start_add(_NCH - 1, 0)
        wait_add(1)
        wait_add(0)
        plsc.subcore_barrier()

        @pl.when(sid == 0)
        def _writeout():
            pltpu.sync_copy(acc, out_hbm.at[cid])

    return k(oe, col4, zeros)


def kernel(x, edge_index, edge_attr, params):
    row4 = edge_index[0].reshape(_NC, _NS, _NCH, _C)
    col4 = edge_index[1].reshape(_NC, _NS, _NCH, _C)

    em = params["edge_model"]
    nm = params["node_model"]
    w_edge = jnp.stack([em["edge"]["w"]]
                       + [l["w"] for l in em["mlp"]["lins"]]
                       + [l["w"] for l in nm["mlp1"]["lins"]])
    b_edge = jnp.stack([em["edge"]["b"]]
                       + [l["b"] for l in em["mlp"]["lins"]]
                       + [l["b"] for l in nm["mlp1"]["lins"]]).reshape(7, 1, _HID)
    w_node = jnp.stack([l["w"] for l in nm["mlp2"]["lins"]])
    b_node = jnp.stack([l["b"] for l in nm["mlp2"]["lins"]]).reshape(3, 1, _HID)
    wij = jnp.stack([em["node_in"]["w"], em["node_out"]["w"]])
    bij = jnp.stack([em["node_in"]["b"], em["node_out"]["b"]]).reshape(2, 1, _HID)
    wei = params["edge_init"]["w"]
    bei = params["edge_init"]["b"].reshape(1, _HID)
    wn = params["node_init"]["w"]
    bn = params["node_init"]["b"].reshape(1, _HID)
    zeros = jnp.zeros((_NN, _HID), _F32)

    h, fi, fj = _prologue_call(x, wn, bn, wij, bij)
    e = None
    for layer in range(_DEPTH):
        gi, gj = _gather_call(fi, fj, row4, col4)
        if layer == 0:
            e, oe = _edge_call(edge_attr, gi, gj, w_edge, b_edge, True, wei, bei)
        else:
            e, oe = _edge_call(e, gi, gj, w_edge, b_edge, False)
        p = _scatter_call(oe, col4, zeros)
        if layer == _DEPTH - 1:
            h = _node_call(h, p[0], p[1], w_node, b_node, None, None, False)
        else:
            h, fi, fj = _node_call(h, p[0], p[1], w_node, b_node, wij, bij, True)
    return (h, e)
